# baseline (speedup 1.0000x reference)
import jax
import jax.numpy as jnp
from jax import lax
from jax.experimental import pallas as pl
from jax.experimental.pallas import tpu as pltpu
from jax.experimental.pallas import tpu_sc as plsc

_TOP_K = 5
_IGNORE_INDEX = -100

_SC_ROWS = 512
_NUM_WORKERS = 32
_LANES = 16


def _acc_block_kernel(tgt_ref, x_ref, num_ref, den_ref):
    x = x_ref[...]
    tgt = tgt_ref[0]
    R, C = x.shape
    col = lax.broadcasted_iota(jnp.int32, (R, C), 1)
    onehot = col == tgt
    t = jnp.sum(jnp.where(onehot, x, 0.0), axis=1, keepdims=True)
    gt = (x > t).astype(jnp.int32)
    eq_lt = ((x == t) & (col < tgt)).astype(jnp.int32)
    cnt = jnp.sum(gt + eq_lt, axis=1)
    not_ign = tgt[:, 0] != _IGNORE_INDEX
    correct = (cnt < _TOP_K) & not_ign
    num = jnp.sum(correct.astype(jnp.float32))
    den = jnp.sum(not_ign.astype(jnp.float32))

    @pl.when(pl.program_id(0) == 0)
    def _():
        num_ref[0, 0] = 0.0
        den_ref[0, 0] = 0.0

    num_ref[0, 0] += num
    den_ref[0, 0] += den


def _tc_part(logits, targets, start_row, rows):
    C = logits.shape[1]
    R = 64
    G = rows // R
    tgt3 = lax.dynamic_slice_in_dim(targets.astype(jnp.int32), start_row,
                                    rows).reshape(G, R, 1)
    blk0 = start_row // R
    num, den = pl.pallas_call(
        _acc_block_kernel,
        grid=(G,),
        in_specs=[
            pl.BlockSpec((1, R, 1), lambda i: (i, 0, 0)),
            pl.BlockSpec((R, C), lambda i: (i + blk0, 0)),
        ],
        out_specs=[
            pl.BlockSpec(memory_space=pltpu.SMEM, block_shape=(1, 1),
                         index_map=lambda i: (0, 0)),
            pl.BlockSpec(memory_space=pltpu.SMEM, block_shape=(1, 1),
                         index_map=lambda i: (0, 0)),
        ],
        out_shape=[
            jax.ShapeDtypeStruct((1, 1), jnp.float32),
            jax.ShapeDtypeStruct((1, 1), jnp.float32),
        ],
    )(tgt3, logits)
    return num[0, 0], den[0, 0]


def _make_sc_part(C):
    rpt = _SC_ROWS // _NUM_WORKERS
    n_bands = rpt // 8
    n_t_full = C // 128
    rem = C - n_t_full * 128
    nt_c = 96
    n_full = n_t_full // nt_c
    nt_tail = n_t_full - n_full * nt_c
    mesh = plsc.VectorSubcoreMesh(core_axis_name="c", subcore_axis_name="s")

    def sc_call(logits, targets_sc):
        @pl.kernel(
            mesh=mesh,
            out_type=[
                jax.ShapeDtypeStruct((_NUM_WORKERS, _LANES), jnp.float32),
                jax.ShapeDtypeStruct((_NUM_WORKERS, _LANES), jnp.float32),
            ],
            scratch_types=[
                pltpu.VMEM((nt_c, 8, 128), jnp.float32),
                pltpu.VMEM((8, 8, 128), jnp.float32),
                pltpu.VMEM((8, max(rem, _LANES)), jnp.float32),
                pltpu.VMEM((rpt + _LANES,), jnp.int32),
                pltpu.VMEM((_LANES,), jnp.float32),
                pltpu.VMEM((_LANES,), jnp.float32),
                pltpu.VMEM((_LANES,), jnp.float32),
                pltpu.SemaphoreType.DMA,
            ],
            compiler_params=pltpu.CompilerParams(needs_layout_passes=False),
        )
        def sc_kernel(logits_hbm, tgt_hbm, num_hbm, den_hbm,
                      chunk_buf, tbuf, pbuf, tgt_buf, tstage, num_buf,
                      den_buf, sem):
            wid = lax.axis_index("s") * 2 + lax.axis_index("c")
            base = wid * rpt
            pltpu.sync_copy(tgt_hbm.at[pl.ds(base, rpt)],
                            tgt_buf.at[pl.ds(0, rpt)])
            lane = lax.iota(jnp.int32, _LANES)
            zero16i = jnp.zeros((_LANES,), jnp.int32)

            def scan_chunk(row0, c0t, nt, t_splats, tgt_splats, ranks):
                copies = [
                    pltpu.async_copy(
                        logits_hbm.at[pl.ds(row0, 8),
                                      pl.ds((c0t + tj) * 128, 128)],
                        chunk_buf.at[tj], sem)
                    for tj in range(nt)
                ]
                for cp in copies:
                    cp.wait()

                def tile_body(tj, rks):
                    col0 = (c0t + tj) * 128
                    out = []
                    for r in range(8):
                        rk = rks[r]
                        for g in range(8):
                            v = chunk_buf[tj, r, pl.ds(g * 16, 16)]
                            col = lane + (col0 + g * 16)
                            pred = (v > t_splats[r]) | (
                                (v == t_splats[r]) & (col < tgt_splats[r]))
                            rk = rk + plsc.all_reduce_population_count(pred)
                        out.append(rk)
                    return tuple(out)

                return lax.fori_loop(0, nt, tile_body, ranks)

            def band_body(b, carry):
                num_v, den_v = carry
                row0 = base + b * 8
                if rem:
                    pltpu.sync_copy(
                        logits_hbm.at[pl.ds(row0, 8),
                                      pl.ds(n_t_full * 128, rem)],
                        pbuf.at[pl.ds(0, 8), pl.ds(0, rem)])
                tgt_band = jnp.maximum(tgt_buf[pl.ds(b * 8, _LANES)], 0)
                tcopies = []
                for r in range(8):
                    tgt_r = jnp.maximum(tgt_buf[pl.ds(b * 8 + r, _LANES)][0],
                                        0)
                    tile_r = jnp.minimum(tgt_r // 128, n_t_full - 1)
                    tcopies.append(pltpu.async_copy(
                        logits_hbm.at[pl.ds(row0, 8),
                                      pl.ds(pl.multiple_of(tile_r * 128, 128),
                                            128)],
                        tbuf.at[r], sem))
                for cp in tcopies:
                    cp.wait()
                slot = jnp.minimum(lane, 7)
                tile_vec = jnp.minimum(tgt_band // 128, n_t_full - 1)
                off128 = jnp.where(
                    lane < 8,
                    jnp.minimum(tgt_band - tile_vec * 128, 127), 0)
                t_vec8 = plsc.load_gather(tbuf, [slot, slot, off128])
                if rem:
                    off_rem = jnp.clip(tgt_band - n_t_full * 128, 0, rem - 1)
                    t_prem = plsc.load_gather(
                        pbuf, [slot, jnp.where(lane < 8, off_rem, 0)])
                    t_vec8 = jnp.where(tgt_band >= n_t_full * 128,
                                       t_prem, t_vec8)
                tstage[...] = t_vec8
                t_splats = []
                tgt_splats = []
                for r in range(8):
                    rfull = jnp.full((_LANES,), r, jnp.int32)
                    t_splats.append(plsc.load_gather(tstage, [rfull]))
                    tgt_splats.append(plsc.load_gather(
                        tgt_buf, [jnp.full((_LANES,), b * 8 + r, jnp.int32)]))

                ranks = tuple(zero16i for _ in range(8))
                ranks = lax.fori_loop(
                    0, n_full,
                    lambda ci, rks: scan_chunk(row0, ci * nt_c, nt_c,
                                               t_splats, tgt_splats, rks),
                    ranks)
                if nt_tail:
                    ranks = scan_chunk(row0, n_full * nt_c, nt_tail,
                                       t_splats, tgt_splats, ranks)
                if rem:
                    new_ranks = []
                    for r in range(8):
                        rk = ranks[r]
                        for g in range(rem // _LANES):
                            v = pbuf[r, pl.ds(g * 16, 16)]
                            col = lane + (n_t_full * 128 + g * 16)
                            pred = (v > t_splats[r]) | (
                                (v == t_splats[r]) & (col < tgt_splats[r]))
                            rk = rk + plsc.all_reduce_population_count(pred)
                        new_ranks.append(rk)
                    ranks = tuple(new_ranks)

                for r in range(8):
                    not_ign = tgt_splats[r] != _IGNORE_INDEX
                    ok = (ranks[r] < _TOP_K) & not_ign
                    num_v = num_v + jnp.where(ok, 1.0, 0.0)
                    den_v = den_v + jnp.where(not_ign, 1.0, 0.0)
                return num_v, den_v

            num_v, den_v = lax.fori_loop(
                0, n_bands, band_body,
                (jnp.zeros((_LANES,), jnp.float32),
                 jnp.zeros((_LANES,), jnp.float32)))
            num_buf[...] = num_v
            den_buf[...] = den_v
            pltpu.sync_copy(num_buf, num_hbm.at[wid])
            pltpu.sync_copy(den_buf, den_hbm.at[wid])

        return sc_kernel(logits, targets_sc)

    return sc_call


def kernel(logits, targets):
    B, C = logits.shape
    targets = targets.astype(jnp.int32)
    sc_call = _make_sc_part(C)
    num_sc2, den_sc2 = sc_call(logits, targets[:_SC_ROWS])
    num_tc, den_tc = _tc_part(logits, targets, _SC_ROWS, B - _SC_ROWS)
    num = num_tc + jnp.sum(num_sc2[:, 0])
    den = den_tc + jnp.sum(den_sc2[:, 0])
    return num / den


# speedup vs baseline: 2.6309x; 2.6309x over previous
import jax
import jax.numpy as jnp
from jax import lax
from jax.experimental import pallas as pl
from jax.experimental.pallas import tpu as pltpu
from jax.experimental.pallas import tpu_sc as plsc

_TOP_K = 5
_IGNORE_INDEX = -100

_SC_ROWS = 512
_NUM_WORKERS = 32
_LANES = 16


def _acc_block_kernel(tgt_ref, x_ref, num_ref, den_ref):
    x = x_ref[...]
    tgt = tgt_ref[0]
    R, C = x.shape
    col = lax.broadcasted_iota(jnp.int32, (R, C), 1)
    onehot = col == tgt
    t = jnp.sum(jnp.where(onehot, x, 0.0), axis=1, keepdims=True)
    gt = (x > t).astype(jnp.int32)
    eq_lt = ((x == t) & (col < tgt)).astype(jnp.int32)
    cnt = jnp.sum(gt + eq_lt, axis=1)
    not_ign = tgt[:, 0] != _IGNORE_INDEX
    correct = (cnt < _TOP_K) & not_ign
    num = jnp.sum(correct.astype(jnp.float32))
    den = jnp.sum(not_ign.astype(jnp.float32))

    @pl.when(pl.program_id(0) == 0)
    def _():
        num_ref[0, 0] = 0.0
        den_ref[0, 0] = 0.0

    num_ref[0, 0] += num
    den_ref[0, 0] += den


def _tc_part(logits, targets, start_row, rows):
    C = logits.shape[1]
    R = 64
    G = rows // R
    tgt3 = lax.dynamic_slice_in_dim(targets.astype(jnp.int32), start_row,
                                    rows).reshape(G, R, 1)
    blk0 = start_row // R
    num, den = pl.pallas_call(
        _acc_block_kernel,
        grid=(G,),
        in_specs=[
            pl.BlockSpec((1, R, 1), lambda i: (i, 0, 0)),
            pl.BlockSpec((R, C), lambda i: (i + blk0, 0)),
        ],
        out_specs=[
            pl.BlockSpec(memory_space=pltpu.SMEM, block_shape=(1, 1),
                         index_map=lambda i: (0, 0)),
            pl.BlockSpec(memory_space=pltpu.SMEM, block_shape=(1, 1),
                         index_map=lambda i: (0, 0)),
        ],
        out_shape=[
            jax.ShapeDtypeStruct((1, 1), jnp.float32),
            jax.ShapeDtypeStruct((1, 1), jnp.float32),
        ],
    )(tgt3, logits)
    return num[0, 0], den[0, 0]


def _make_sc_part(C):
    rpt = _SC_ROWS // _NUM_WORKERS
    n_bands = rpt // 8
    n_t_full = C // 128
    rem = C - n_t_full * 128
    nt_c = 96
    n_full = n_t_full // nt_c
    nt_tail = n_t_full - n_full * nt_c
    mesh = plsc.VectorSubcoreMesh(core_axis_name="c", subcore_axis_name="s")

    def sc_call(logits, targets_sc):
        @pl.kernel(
            mesh=mesh,
            out_type=[
                jax.ShapeDtypeStruct((_NUM_WORKERS, _LANES), jnp.float32),
                jax.ShapeDtypeStruct((_NUM_WORKERS, _LANES), jnp.float32),
            ],
            scratch_types=[
                pltpu.VMEM((nt_c, 8, 128), jnp.float32),
                pltpu.VMEM((8, 8, 128), jnp.float32),
                pltpu.VMEM((8, max(rem, _LANES)), jnp.float32),
                pltpu.VMEM((rpt + _LANES,), jnp.int32),
                pltpu.VMEM((_LANES,), jnp.float32),
                pltpu.VMEM((_LANES,), jnp.float32),
                pltpu.VMEM((_LANES,), jnp.float32),
                pltpu.SemaphoreType.DMA,
            ],
            compiler_params=pltpu.CompilerParams(needs_layout_passes=False),
        )
        def sc_kernel(logits_hbm, tgt_hbm, num_hbm, den_hbm,
                      chunk_buf, tbuf, pbuf, tgt_buf, tstage, num_buf,
                      den_buf, sem):
            wid = lax.axis_index("s") * 2 + lax.axis_index("c")
            base = wid * rpt
            pltpu.sync_copy(tgt_hbm.at[pl.ds(base, rpt)],
                            tgt_buf.at[pl.ds(0, rpt)])
            lane = lax.iota(jnp.int32, _LANES)
            zero16i = jnp.zeros((_LANES,), jnp.int32)

            def scan_chunk(row0, c0t, nt, t_splats, tgt_splats, ranks):
                copies = [
                    pltpu.async_copy(
                        logits_hbm.at[pl.ds(row0, 8),
                                      pl.ds((c0t + tj) * 128, 128)],
                        chunk_buf.at[tj], sem)
                    for tj in range(nt)
                ]
                for cp in copies:
                    cp.wait()

                out = []
                for r in range(8):
                    t_r = t_splats[r]
                    tgt_r = tgt_splats[r]

                    def tile_body(tj, rk, t_r=t_r, tgt_r=tgt_r, r=r):
                        col0 = (c0t + tj) * 128
                        for g in range(8):
                            v = chunk_buf[tj, r, pl.ds(g * 16, 16)]
                            col = lane + (col0 + g * 16)
                            pred = jnp.where(col < tgt_r, v >= t_r, v > t_r)
                            rk = rk + plsc.all_reduce_population_count(pred)
                        return rk

                    out.append(lax.fori_loop(0, nt, tile_body, ranks[r]))
                return tuple(out)

            def band_body(b, carry):
                num_v, den_v = carry
                row0 = base + b * 8
                if rem:
                    pltpu.sync_copy(
                        logits_hbm.at[pl.ds(row0, 8),
                                      pl.ds(n_t_full * 128, rem)],
                        pbuf.at[pl.ds(0, 8), pl.ds(0, rem)])
                tgt_band = jnp.maximum(tgt_buf[pl.ds(b * 8, _LANES)], 0)
                tcopies = []
                for r in range(8):
                    tgt_r = jnp.maximum(tgt_buf[pl.ds(b * 8 + r, _LANES)][0],
                                        0)
                    tile_r = jnp.minimum(tgt_r // 128, n_t_full - 1)
                    tcopies.append(pltpu.async_copy(
                        logits_hbm.at[pl.ds(row0, 8),
                                      pl.ds(pl.multiple_of(tile_r * 128, 128),
                                            128)],
                        tbuf.at[r], sem))
                for cp in tcopies:
                    cp.wait()
                slot = jnp.minimum(lane, 7)
                tile_vec = jnp.minimum(tgt_band // 128, n_t_full - 1)
                off128 = jnp.where(
                    lane < 8,
                    jnp.minimum(tgt_band - tile_vec * 128, 127), 0)
                t_vec8 = plsc.load_gather(tbuf, [slot, slot, off128])
                if rem:
                    off_rem = jnp.clip(tgt_band - n_t_full * 128, 0, rem - 1)
                    t_prem = plsc.load_gather(
                        pbuf, [slot, jnp.where(lane < 8, off_rem, 0)])
                    t_vec8 = jnp.where(tgt_band >= n_t_full * 128,
                                       t_prem, t_vec8)
                tstage[...] = t_vec8
                t_splats = []
                tgt_splats = []
                for r in range(8):
                    rfull = jnp.full((_LANES,), r, jnp.int32)
                    t_splats.append(plsc.load_gather(tstage, [rfull]))
                    tgt_splats.append(plsc.load_gather(
                        tgt_buf, [jnp.full((_LANES,), b * 8 + r, jnp.int32)]))

                ranks = tuple(zero16i for _ in range(8))
                ranks = lax.fori_loop(
                    0, n_full,
                    lambda ci, rks: scan_chunk(row0, ci * nt_c, nt_c,
                                               t_splats, tgt_splats, rks),
                    ranks)
                if nt_tail:
                    ranks = scan_chunk(row0, n_full * nt_c, nt_tail,
                                       t_splats, tgt_splats, ranks)
                if rem:
                    new_ranks = []
                    for r in range(8):
                        rk = ranks[r]
                        for g in range(rem // _LANES):
                            v = pbuf[r, pl.ds(g * 16, 16)]
                            col = lane + (n_t_full * 128 + g * 16)
                            pred = (v > t_splats[r]) | (
                                (v == t_splats[r]) & (col < tgt_splats[r]))
                            rk = rk + plsc.all_reduce_population_count(pred)
                        new_ranks.append(rk)
                    ranks = tuple(new_ranks)

                for r in range(8):
                    not_ign = tgt_splats[r] != _IGNORE_INDEX
                    ok = (ranks[r] < _TOP_K) & not_ign
                    num_v = num_v + jnp.where(ok, 1.0, 0.0)
                    den_v = den_v + jnp.where(not_ign, 1.0, 0.0)
                return num_v, den_v

            num_v, den_v = lax.fori_loop(
                0, n_bands, band_body,
                (jnp.zeros((_LANES,), jnp.float32),
                 jnp.zeros((_LANES,), jnp.float32)))
            num_buf[...] = num_v
            den_buf[...] = den_v
            pltpu.sync_copy(num_buf, num_hbm.at[wid])
            pltpu.sync_copy(den_buf, den_hbm.at[wid])

        return sc_kernel(logits, targets_sc)

    return sc_call


def kernel(logits, targets):
    B, C = logits.shape
    targets = targets.astype(jnp.int32)
    sc_call = _make_sc_part(C)
    num_sc2, den_sc2 = sc_call(logits, targets[:_SC_ROWS])
    num_tc, den_tc = _tc_part(logits, targets, _SC_ROWS, B - _SC_ROWS)
    num = num_tc + jnp.sum(num_sc2[:, 0])
    den = den_tc + jnp.sum(den_sc2[:, 0])
    return num / den


# speedup vs baseline: 3.5167x; 1.3367x over previous
import jax
import jax.numpy as jnp
from jax import lax
from jax.experimental import pallas as pl
from jax.experimental.pallas import tpu as pltpu
from jax.experimental.pallas import tpu_sc as plsc

_TOP_K = 5
_IGNORE_INDEX = -100

_SC_ROWS = 256
_NUM_WORKERS = 32
_LANES = 16


def _acc_block_kernel(tgt_ref, x_ref, num_ref, den_ref):
    x = x_ref[...]
    tgt = tgt_ref[0]
    R, C = x.shape
    col = lax.broadcasted_iota(jnp.int32, (R, C), 1)
    onehot = col == tgt
    t = jnp.sum(jnp.where(onehot, x, 0.0), axis=1, keepdims=True)
    gt = (x > t).astype(jnp.int32)
    eq_lt = ((x == t) & (col < tgt)).astype(jnp.int32)
    cnt = jnp.sum(gt + eq_lt, axis=1)
    not_ign = tgt[:, 0] != _IGNORE_INDEX
    correct = (cnt < _TOP_K) & not_ign
    num = jnp.sum(correct.astype(jnp.float32))
    den = jnp.sum(not_ign.astype(jnp.float32))

    @pl.when(pl.program_id(0) == 0)
    def _():
        num_ref[0, 0] = 0.0
        den_ref[0, 0] = 0.0

    num_ref[0, 0] += num
    den_ref[0, 0] += den


def _tc_part(logits, targets, start_row, rows):
    C = logits.shape[1]
    R = 64
    G = rows // R
    tgt3 = lax.dynamic_slice_in_dim(targets.astype(jnp.int32), start_row,
                                    rows).reshape(G, R, 1)
    blk0 = start_row // R
    num, den = pl.pallas_call(
        _acc_block_kernel,
        grid=(G,),
        in_specs=[
            pl.BlockSpec((1, R, 1), lambda i: (i, 0, 0)),
            pl.BlockSpec((R, C), lambda i: (i + blk0, 0)),
        ],
        out_specs=[
            pl.BlockSpec(memory_space=pltpu.SMEM, block_shape=(1, 1),
                         index_map=lambda i: (0, 0)),
            pl.BlockSpec(memory_space=pltpu.SMEM, block_shape=(1, 1),
                         index_map=lambda i: (0, 0)),
        ],
        out_shape=[
            jax.ShapeDtypeStruct((1, 1), jnp.float32),
            jax.ShapeDtypeStruct((1, 1), jnp.float32),
        ],
    )(tgt3, logits)
    return num[0, 0], den[0, 0]


def _make_sc_part(C):
    rpt = _SC_ROWS // _NUM_WORKERS
    n_bands = rpt // 8
    n_t_full = C // 128
    rem = C - n_t_full * 128
    nt_c = 96
    n_full = n_t_full // nt_c
    nt_tail = n_t_full - n_full * nt_c
    mesh = plsc.VectorSubcoreMesh(core_axis_name="c", subcore_axis_name="s")

    def sc_call(logits, targets_sc):
        @pl.kernel(
            mesh=mesh,
            out_type=[
                jax.ShapeDtypeStruct((_NUM_WORKERS, _LANES), jnp.float32),
                jax.ShapeDtypeStruct((_NUM_WORKERS, _LANES), jnp.float32),
            ],
            scratch_types=[
                pltpu.VMEM((nt_c, 8, 128), jnp.float32),
                pltpu.VMEM((8, 8, 128), jnp.float32),
                pltpu.VMEM((8, max(rem, _LANES)), jnp.float32),
                pltpu.VMEM((rpt + _LANES,), jnp.int32),
                pltpu.VMEM((_LANES,), jnp.float32),
                pltpu.VMEM((_LANES,), jnp.float32),
                pltpu.VMEM((_LANES,), jnp.float32),
                pltpu.SemaphoreType.DMA,
            ],
            compiler_params=pltpu.CompilerParams(needs_layout_passes=False),
        )
        def sc_kernel(logits_hbm, tgt_hbm, num_hbm, den_hbm,
                      chunk_buf, tbuf, pbuf, tgt_buf, tstage, num_buf,
                      den_buf, sem):
            wid = lax.axis_index("s") * 2 + lax.axis_index("c")
            base = wid * rpt
            pltpu.sync_copy(tgt_hbm.at[pl.ds(base, rpt)],
                            tgt_buf.at[pl.ds(0, rpt)])
            lane = lax.iota(jnp.int32, _LANES)
            zero16i = jnp.zeros((_LANES,), jnp.int32)

            def scan_chunk(row0, c0t, nt, t_splats, tgt_splats, ranks):
                copies = [
                    pltpu.async_copy(
                        logits_hbm.at[pl.ds(row0, 8),
                                      pl.ds((c0t + tj) * 128, 128)],
                        chunk_buf.at[tj], sem)
                    for tj in range(nt)
                ]
                for cp in copies:
                    cp.wait()

                out = []
                for r in range(8):
                    t_r = t_splats[r]
                    tgt_r = tgt_splats[r]

                    def tile_body(tj, rk, t_r=t_r, tgt_r=tgt_r, r=r):
                        col0 = (c0t + tj) * 128
                        for g in range(8):
                            v = chunk_buf[tj, r, pl.ds(g * 16, 16)]
                            col = lane + (col0 + g * 16)
                            pred = jnp.where(col < tgt_r, v >= t_r, v > t_r)
                            rk = rk + plsc.all_reduce_population_count(pred)
                        return rk

                    out.append(lax.fori_loop(0, nt, tile_body, ranks[r]))
                return tuple(out)

            def band_body(b, carry):
                num_v, den_v = carry
                row0 = base + b * 8
                if rem:
                    pltpu.sync_copy(
                        logits_hbm.at[pl.ds(row0, 8),
                                      pl.ds(n_t_full * 128, rem)],
                        pbuf.at[pl.ds(0, 8), pl.ds(0, rem)])
                tgt_band = jnp.maximum(tgt_buf[pl.ds(b * 8, _LANES)], 0)
                tcopies = []
                for r in range(8):
                    tgt_r = jnp.maximum(tgt_buf[pl.ds(b * 8 + r, _LANES)][0],
                                        0)
                    tile_r = jnp.minimum(tgt_r // 128, n_t_full - 1)
                    tcopies.append(pltpu.async_copy(
                        logits_hbm.at[pl.ds(row0, 8),
                                      pl.ds(pl.multiple_of(tile_r * 128, 128),
                                            128)],
                        tbuf.at[r], sem))
                for cp in tcopies:
                    cp.wait()
                slot = jnp.minimum(lane, 7)
                tile_vec = jnp.minimum(tgt_band // 128, n_t_full - 1)
                off128 = jnp.where(
                    lane < 8,
                    jnp.minimum(tgt_band - tile_vec * 128, 127), 0)
                t_vec8 = plsc.load_gather(tbuf, [slot, slot, off128])
                if rem:
                    off_rem = jnp.clip(tgt_band - n_t_full * 128, 0, rem - 1)
                    t_prem = plsc.load_gather(
                        pbuf, [slot, jnp.where(lane < 8, off_rem, 0)])
                    t_vec8 = jnp.where(tgt_band >= n_t_full * 128,
                                       t_prem, t_vec8)
                tstage[...] = t_vec8
                t_splats = []
                tgt_splats = []
                for r in range(8):
                    rfull = jnp.full((_LANES,), r, jnp.int32)
                    t_splats.append(plsc.load_gather(tstage, [rfull]))
                    tgt_splats.append(plsc.load_gather(
                        tgt_buf, [jnp.full((_LANES,), b * 8 + r, jnp.int32)]))

                ranks = tuple(zero16i for _ in range(8))
                ranks = lax.fori_loop(
                    0, n_full,
                    lambda ci, rks: scan_chunk(row0, ci * nt_c, nt_c,
                                               t_splats, tgt_splats, rks),
                    ranks)
                if nt_tail:
                    ranks = scan_chunk(row0, n_full * nt_c, nt_tail,
                                       t_splats, tgt_splats, ranks)
                if rem:
                    new_ranks = []
                    for r in range(8):
                        rk = ranks[r]
                        for g in range(rem // _LANES):
                            v = pbuf[r, pl.ds(g * 16, 16)]
                            col = lane + (n_t_full * 128 + g * 16)
                            pred = (v > t_splats[r]) | (
                                (v == t_splats[r]) & (col < tgt_splats[r]))
                            rk = rk + plsc.all_reduce_population_count(pred)
                        new_ranks.append(rk)
                    ranks = tuple(new_ranks)

                for r in range(8):
                    not_ign = tgt_splats[r] != _IGNORE_INDEX
                    ok = (ranks[r] < _TOP_K) & not_ign
                    num_v = num_v + jnp.where(ok, 1.0, 0.0)
                    den_v = den_v + jnp.where(not_ign, 1.0, 0.0)
                return num_v, den_v

            num_v, den_v = lax.fori_loop(
                0, n_bands, band_body,
                (jnp.zeros((_LANES,), jnp.float32),
                 jnp.zeros((_LANES,), jnp.float32)))
            num_buf[...] = num_v
            den_buf[...] = den_v
            pltpu.sync_copy(num_buf, num_hbm.at[wid])
            pltpu.sync_copy(den_buf, den_hbm.at[wid])

        return sc_kernel(logits, targets_sc)

    return sc_call


def kernel(logits, targets):
    B, C = logits.shape
    targets = targets.astype(jnp.int32)
    sc_call = _make_sc_part(C)
    num_sc2, den_sc2 = sc_call(logits, targets[:_SC_ROWS])
    num_tc, den_tc = _tc_part(logits, targets, _SC_ROWS, B - _SC_ROWS)
    num = num_tc + jnp.sum(num_sc2[:, 0])
    den = den_tc + jnp.sum(den_sc2[:, 0])
    return num / den
